# Initial kernel scaffold; baseline (speedup 1.0000x reference)
import jax
import jax.numpy as jnp
from jax.experimental import pallas as pl


def kernel(atom_types, edge_index, distance, emb_table, eu_W1, eu_b1, eu_W2, eu_b2, pn1_W, pn1_b, pe_W1, pe_b1, pe_W2, pe_b2, pn2_W1, pn2_b1, pn2_W2, pn2_b2, d1_W, d1_b, d2_W, d2_b):
    raise NotImplementedError("write your pallas kernel here")


# speedup vs baseline: 1.9107x; 1.9107x over previous
import functools

import jax
import jax.numpy as jnp
from jax import lax
from jax.experimental import pallas as pl
from jax.experimental.pallas import tpu as pltpu
from jax.experimental.pallas import tpu_sc as plsc

N = 50000
E = 800000
DIM = 64
RBF = 64
NCONV = 3
CUTOFF = 5.0
NTYPES = 100

NC = 2
NS = 16
NW = NC * NS

_LOG2 = 0.6931471805599453


def _ssp(x):
    return jnp.logaddexp(x, 0.0) - _LOG2


def _mesh():
    return plsc.VectorSubcoreMesh(
        core_axis_name="c", subcore_axis_name="s", num_cores=NC, num_subcores=NS
    )


EW = E // NW
GK = 128
GFULL = EW // GK
GTAIL = EW - GFULL * GK
KG = 5
NGRP = GFULL // KG


def _gather_body(nf, src, dst, hsrc, hdst, ibs, ibd, rbs, rbd, isem, gsem, wsem):
    wid = lax.axis_index("s") * NC + lax.axis_index("c")
    base0 = wid * EW

    def do_group(goff, sizes):
        cps = []
        for k, sz in enumerate(sizes):
            off = goff + k * GK
            cps.append(pltpu.async_copy(src.at[pl.ds(off, sz)],
                                        ibs.at[k, pl.ds(0, sz)], isem))
            cps.append(pltpu.async_copy(dst.at[pl.ds(off, sz)],
                                        ibd.at[k, pl.ds(0, sz)], isem))
        for cp in cps:
            cp.wait()
        cps = []
        for k, sz in enumerate(sizes):
            cps.append(pltpu.async_copy(nf.at[ibs.at[k, pl.ds(0, sz)]],
                                        rbs.at[k, pl.ds(0, sz)], gsem))
            cps.append(pltpu.async_copy(nf.at[ibd.at[k, pl.ds(0, sz)]],
                                        rbd.at[k, pl.ds(0, sz)], gsem))
        for cp in cps:
            cp.wait()
        cps = []
        for k, sz in enumerate(sizes):
            off = goff + k * GK
            cps.append(pltpu.async_copy(rbs.at[k, pl.ds(0, sz)],
                                        hsrc.at[pl.ds(off, sz)], wsem))
            cps.append(pltpu.async_copy(rbd.at[k, pl.ds(0, sz)],
                                        hdst.at[pl.ds(off, sz)], wsem))
        for cp in cps:
            cp.wait()

    def grp(g, _):
        do_group(base0 + g * (KG * GK), [GK] * KG)
        return _

    lax.fori_loop(0, NGRP, grp, 0)
    do_group(base0 + NGRP * KG * GK, [GTAIL])


def _sc_gather(nf, src, dst):
    f = pl.kernel(
        _gather_body,
        out_type=[
            jax.ShapeDtypeStruct((E, DIM), jnp.float32),
            jax.ShapeDtypeStruct((E, DIM), jnp.float32),
        ],
        mesh=_mesh(),
        scratch_types=[
            pltpu.VMEM((KG, GK), jnp.int32),
            pltpu.VMEM((KG, GK), jnp.int32),
            pltpu.VMEM((KG, GK, DIM), jnp.float32),
            pltpu.VMEM((KG, GK, DIM), jnp.float32),
            pltpu.SemaphoreType.DMA,
            pltpu.SemaphoreType.DMA,
            pltpu.SemaphoreType.DMA,
        ],
        compiler_params=pltpu.CompilerParams(use_tc_tiling_on_sc=False),
    )
    return f(nf, src, dst)


HALF = N // NC
ECH = E // NS
SFULL = ECH // GK
STAIL = ECH - SFULL * GK
SKG = 3
SNGRP = SFULL // SKG
ZR = 1568
DUMP = NS * ZR
ACC_R = DUMP + 8
WOUT = 1560


def _scatter_body(he, dst, zeros, h_out, dbuf, lb0, lb1, lb2,
                  rbuf, acc, isem, gsem, ssem):
    c = lax.axis_index("c")
    t = lax.axis_index("s")
    lo = c * HALF
    lbufs = [lb0, lb1, lb2]

    pltpu.sync_copy(zeros, acc.at[pl.ds(t * ZR, ZR)])
    plsc.subcore_barrier()

    base0 = t * ECH

    def do_group(goff, sizes):
        cps = []
        for k, sz in enumerate(sizes):
            off = goff + k * GK
            cps.append(pltpu.async_copy(he.at[pl.ds(off, sz)],
                                        rbuf.at[k, pl.ds(0, sz)], gsem))
            cps.append(pltpu.async_copy(dst.at[pl.ds(off, sz)],
                                        dbuf.at[k, pl.ds(0, sz)], isem))
        for cp in cps:
            cp.wait()
        for k, sz in enumerate(sizes):
            for j in range(sz // 16):
                v = dbuf[k, pl.ds(j * 16, 16)]
                inb = (v >= lo) & (v < lo + HALF)
                lbufs[k][pl.ds(j * 16, 16)] = jnp.where(inb, v - lo, DUMP)
        cps = []
        for k, sz in enumerate(sizes):
            cps.append(pltpu.async_copy(rbuf.at[k, pl.ds(0, sz)],
                                        acc.at[lbufs[k].at[pl.ds(0, sz)]],
                                        ssem, add=True))
        for cp in cps:
            cp.wait()

    def grp(g, _):
        do_group(base0 + g * (SKG * GK), [GK] * SKG)
        return _

    lax.fori_loop(0, SNGRP, grp, 0)
    do_group(base0 + SNGRP * SKG * GK, [STAIL])

    plsc.subcore_barrier()
    pltpu.sync_copy(acc.at[pl.ds(t * WOUT, WOUT)],
                    h_out.at[pl.ds(lo + t * WOUT, WOUT)])

    @pl.when(t == 0)
    def _():
        pltpu.sync_copy(acc.at[pl.ds(NS * WOUT, HALF - NS * WOUT)],
                        h_out.at[pl.ds(lo + NS * WOUT, HALF - NS * WOUT)])


def _sc_scatter(he, dst, zeros):
    f = pl.kernel(
        _scatter_body,
        out_type=jax.ShapeDtypeStruct((N, DIM), jnp.float32),
        mesh=_mesh(),
        scratch_types=[
            pltpu.VMEM((SKG, GK), jnp.int32),
            pltpu.VMEM((GK,), jnp.int32),
            pltpu.VMEM((GK,), jnp.int32),
            pltpu.VMEM((GK,), jnp.int32),
            pltpu.VMEM((SKG, GK, DIM), jnp.float32),
            pltpu.VMEM_SHARED((ACC_R, DIM), jnp.float32),
            pltpu.SemaphoreType.DMA,
            pltpu.SemaphoreType.DMA,
            pltpu.SemaphoreType.DMA,
        ],
        compiler_params=pltpu.CompilerParams(use_tc_tiling_on_sc=False),
    )
    return f(he, dst, zeros)


BN = 2048
BE = 2048
NTP = 128


def _embed_body(at_ref, emb_ref, out_ref):
    at = at_ref[...]
    tid = lax.broadcasted_iota(jnp.int32, (BN, NTP), 1)
    onehot = jnp.where(at == tid, 1.0, 0.0).astype(jnp.float32)
    out_ref[...] = jnp.dot(onehot, emb_ref[...],
                           preferred_element_type=jnp.float32)


def _tc_embed(atom_types, emb_pad):
    return pl.pallas_call(
        _embed_body,
        grid=(pl.cdiv(N, BN),),
        in_specs=[
            pl.BlockSpec((BN, 1), lambda i: (i, 0)),
            pl.BlockSpec((NTP, DIM), lambda i: (0, 0)),
        ],
        out_specs=pl.BlockSpec((BN, DIM), lambda i: (i, 0)),
        out_shape=jax.ShapeDtypeStruct((N, DIM), jnp.float32),
    )(atom_types.reshape(N, 1), emb_pad)


def _edge_body(first, last, hs_ref, hd_ref, ef_ref, w1_ref, b1_ref, w2_ref,
               b2_ref, pw1_ref, pb1_ref, pw2_ref, pb2_ref, *out_refs):
    if first:
        d = ef_ref[...]
        step = CUTOFF / (RBF - 1)
        offs = lax.broadcasted_iota(jnp.int32, (1, RBF), 1).astype(jnp.float32) * step
        coeff = -0.5 / (step * step)
        ef = jnp.exp(coeff * (d - offs) ** 2)
    else:
        ef = ef_ref[...]
    w1 = w1_ref[...]
    t = jnp.dot(hs_ref[...], w1[:DIM], preferred_element_type=jnp.float32)
    t += jnp.dot(hd_ref[...], w1[DIM:2 * DIM], preferred_element_type=jnp.float32)
    t += jnp.dot(ef, w1[2 * DIM:], preferred_element_type=jnp.float32)
    t = _ssp(t + b1_ref[...])
    ef_new = jnp.dot(t, w2_ref[...], preferred_element_type=jnp.float32) + b2_ref[...]
    u = _ssp(jnp.dot(ef_new, pw1_ref[...], preferred_element_type=jnp.float32)
             + pb1_ref[...])
    he = jnp.dot(u, pw2_ref[...], preferred_element_type=jnp.float32) + pb2_ref[...]
    if last:
        out_refs[0][...] = he
    else:
        out_refs[0][...] = ef_new
        out_refs[1][...] = he


def _tc_edge(hs, hd, ef, w1, b1, w2, b2, pw1, pb1, pw2, pb2, first, last):
    efw = 1 if first else RBF
    wspec = lambda r, c: pl.BlockSpec((r, c), lambda i: (0, 0))
    outs = [jax.ShapeDtypeStruct((E, RBF), jnp.float32)]
    out_specs = [pl.BlockSpec((BE, RBF), lambda i: (i, 0))]
    if not last:
        outs.append(jax.ShapeDtypeStruct((E, RBF), jnp.float32))
        out_specs.append(pl.BlockSpec((BE, RBF), lambda i: (i, 0)))
    res = pl.pallas_call(
        functools.partial(_edge_body, first, last),
        grid=(pl.cdiv(E, BE),),
        in_specs=[
            pl.BlockSpec((BE, DIM), lambda i: (i, 0)),
            pl.BlockSpec((BE, DIM), lambda i: (i, 0)),
            pl.BlockSpec((BE, efw), lambda i: (i, 0)),
            wspec(2 * DIM + RBF, 2 * DIM),
            wspec(1, 2 * DIM),
            wspec(2 * DIM, RBF),
            wspec(1, RBF),
            wspec(RBF, DIM),
            wspec(1, DIM),
            wspec(DIM, DIM),
            wspec(1, DIM),
        ],
        out_specs=out_specs,
        out_shape=outs,
    )(hs, hd, ef, w1, b1, w2, b2, pw1, pb1, pw2, pb2)
    if last:
        return None, res[0]
    return res[0], res[1]


def _node_body(h_ref, nf_ref, w1_ref, b1_ref, w2_ref, b2_ref, out_ref):
    t = _ssp(jnp.dot(h_ref[...], w1_ref[...], preferred_element_type=jnp.float32)
             + b1_ref[...])
    out_ref[...] = (nf_ref[...] + b2_ref[...]
                    + jnp.dot(t, w2_ref[...], preferred_element_type=jnp.float32))


def _tc_node(h, nf, w1, b1, w2, b2):
    wspec = lambda r, c: pl.BlockSpec((r, c), lambda i: (0, 0))
    return pl.pallas_call(
        _node_body,
        grid=(pl.cdiv(N, BN),),
        in_specs=[
            pl.BlockSpec((BN, DIM), lambda i: (i, 0)),
            pl.BlockSpec((BN, DIM), lambda i: (i, 0)),
            wspec(DIM, DIM), wspec(1, DIM), wspec(DIM, DIM), wspec(1, DIM),
        ],
        out_specs=pl.BlockSpec((BN, DIM), lambda i: (i, 0)),
        out_shape=jax.ShapeDtypeStruct((N, DIM), jnp.float32),
    )(h, nf, w1, b1, w2, b2)


def _node_last_body(ngrid, h_ref, nf_ref, w1_ref, b1_ref, w2_ref, b2_ref,
                    d1w_ref, d1b_ref, d2w_ref, d2b_ref, out_ref):
    i = pl.program_id(0)
    t = _ssp(jnp.dot(h_ref[...], w1_ref[...], preferred_element_type=jnp.float32)
             + b1_ref[...])
    nf = (nf_ref[...] + b2_ref[...]
          + jnp.dot(t, w2_ref[...], preferred_element_type=jnp.float32))
    atom = _ssp(jnp.dot(nf, d1w_ref[...], preferred_element_type=jnp.float32)
                + d1b_ref[...])
    row = i * BN + lax.broadcasted_iota(jnp.int32, (BN, 1), 0)
    atom = jnp.where(row < N, atom, 0.0)
    partial = jnp.sum(atom * d2w_ref[...])

    @pl.when(i == 0)
    def _():
        out_ref[...] = jnp.zeros_like(out_ref)

    out_ref[...] += partial.reshape(1, 1)

    @pl.when(i == ngrid - 1)
    def _():
        out_ref[...] = out_ref[...] * (1.0 / N) + d2b_ref[...]


def _tc_node_last(h, nf, w1, b1, w2, b2, d1w, d1b, d2w, d2b):
    ngrid = pl.cdiv(N, BN)
    wspec = lambda r, c: pl.BlockSpec((r, c), lambda i: (0, 0))
    return pl.pallas_call(
        functools.partial(_node_last_body, ngrid),
        grid=(ngrid,),
        in_specs=[
            pl.BlockSpec((BN, DIM), lambda i: (i, 0)),
            pl.BlockSpec((BN, DIM), lambda i: (i, 0)),
            wspec(DIM, DIM), wspec(1, DIM), wspec(DIM, DIM), wspec(1, DIM),
            wspec(DIM, DIM // 2), wspec(1, DIM // 2),
            wspec(1, DIM // 2), wspec(1, 1),
        ],
        out_specs=pl.BlockSpec((1, 1), lambda i: (0, 0)),
        out_shape=jax.ShapeDtypeStruct((1, 1), jnp.float32),
    )(h, nf, w1, b1, w2, b2, d1w, d1b, d2w, d2b)


def kernel(atom_types, edge_index, distance, emb_table, eu_W1, eu_b1, eu_W2,
           eu_b2, pn1_W, pn1_b, pe_W1, pe_b1, pe_W2, pe_b2, pn2_W1, pn2_b1,
           pn2_W2, pn2_b2, d1_W, d1_b, d2_W, d2_b):
    src = edge_index[0]
    dst = edge_index[1]
    dist = distance.reshape(E, 1)
    emb_pad = jnp.pad(emb_table, ((0, NTP - NTYPES), (0, 0)))
    zeros = jnp.zeros((ZR, DIM), jnp.float32)

    nf = _tc_embed(atom_types, emb_pad)
    ef = dist
    for i in range(NCONV):
        first = i == 0
        last = i == NCONV - 1
        hs, hd = _sc_gather(nf, src, dst)
        ef, he = _tc_edge(hs, hd, ef, eu_W1[i], eu_b1[i].reshape(1, -1),
                          eu_W2[i], eu_b2[i].reshape(1, -1),
                          pe_W1[i], pe_b1[i].reshape(1, -1),
                          pe_W2[i], pe_b2[i].reshape(1, -1), first, last)
        h = _sc_scatter(he, dst, zeros)
        if not last:
            nf = _tc_node(h, nf, pn2_W1[i], pn2_b1[i].reshape(1, -1),
                          pn2_W2[i], pn2_b2[i].reshape(1, -1))
        else:
            out = _tc_node_last(h, nf, pn2_W1[i], pn2_b1[i].reshape(1, -1),
                                pn2_W2[i], pn2_b2[i].reshape(1, -1),
                                d1_W, d1_b.reshape(1, -1),
                                d2_W.reshape(1, -1), d2_b.reshape(1, 1))
    return out


# speedup vs baseline: 2.1477x; 1.1240x over previous
import functools

import jax
import jax.numpy as jnp
from jax import lax
from jax.experimental import pallas as pl
from jax.experimental.pallas import tpu as pltpu
from jax.experimental.pallas import tpu_sc as plsc

N = 50000
E = 800000
DIM = 64
RBF = 64
NCONV = 3
CUTOFF = 5.0
NTYPES = 100

NC = 2
NS = 16
NW = NC * NS

_LOG2 = 0.6931471805599453


def _ssp(x):
    return jnp.logaddexp(x, 0.0) - _LOG2


def _mesh():
    return plsc.VectorSubcoreMesh(
        core_axis_name="c", subcore_axis_name="s", num_cores=NC, num_subcores=NS
    )


EW = E // NW
GK = 128
QGK = 64
QKG = 3
QFULL = EW // QGK
QTAIL = EW - QFULL * QGK
QNGRP = QFULL // QKG
HID = 2 * DIM


def _gather_body(q1, q2, src, dst, qs, qd, ibs, ibd, rbs, rbd,
                 isem, gsem, wsem):
    wid = lax.axis_index("s") * NC + lax.axis_index("c")
    base0 = wid * EW

    def do_group(goff, sizes):
        cps = []
        for k, sz in enumerate(sizes):
            off = goff + k * QGK
            cps.append(pltpu.async_copy(src.at[pl.ds(off, sz)],
                                        ibs.at[k, pl.ds(0, sz)], isem))
            cps.append(pltpu.async_copy(dst.at[pl.ds(off, sz)],
                                        ibd.at[k, pl.ds(0, sz)], isem))
        for cp in cps:
            cp.wait()
        cps = []
        for k, sz in enumerate(sizes):
            cps.append(pltpu.async_copy(q1.at[ibs.at[k, pl.ds(0, sz)]],
                                        rbs.at[k, pl.ds(0, sz)], gsem))
            cps.append(pltpu.async_copy(q2.at[ibd.at[k, pl.ds(0, sz)]],
                                        rbd.at[k, pl.ds(0, sz)], gsem))
        for cp in cps:
            cp.wait()
        cps = []
        for k, sz in enumerate(sizes):
            off = goff + k * QGK
            cps.append(pltpu.async_copy(rbs.at[k, pl.ds(0, sz)],
                                        qs.at[pl.ds(off, sz)], wsem))
            cps.append(pltpu.async_copy(rbd.at[k, pl.ds(0, sz)],
                                        qd.at[pl.ds(off, sz)], wsem))
        for cp in cps:
            cp.wait()

    def grp(g, _):
        do_group(base0 + g * (QKG * QGK), [QGK] * QKG)
        return _

    lax.fori_loop(0, QNGRP, grp, 0)
    do_group(base0 + QNGRP * QKG * QGK, [QTAIL])


def _sc_gather(q1, q2, src, dst):
    f = pl.kernel(
        _gather_body,
        out_type=[
            jax.ShapeDtypeStruct((E, HID), jnp.float32),
            jax.ShapeDtypeStruct((E, HID), jnp.float32),
        ],
        mesh=_mesh(),
        scratch_types=[
            pltpu.VMEM((QKG, QGK), jnp.int32),
            pltpu.VMEM((QKG, QGK), jnp.int32),
            pltpu.VMEM((QKG, QGK, HID), jnp.float32),
            pltpu.VMEM((QKG, QGK, HID), jnp.float32),
            pltpu.SemaphoreType.DMA,
            pltpu.SemaphoreType.DMA,
            pltpu.SemaphoreType.DMA,
        ],
    )
    return f(q1, q2, src, dst)


HALF = N // NC
ECH = E // NS
SFULL = ECH // GK
STAIL = ECH - SFULL * GK
SKG = 3
SNGRP = SFULL // SKG
ZR = 1568
DUMP = NS * ZR
ACC_R = DUMP + 8
WOUT = 1560


def _scatter_body(he, dst, zeros, h_out, dbuf, lb0, lb1, lb2,
                  rbuf, acc, isem, gsem, ssem):
    c = lax.axis_index("c")
    t = lax.axis_index("s")
    lo = c * HALF
    lbufs = [lb0, lb1, lb2]

    pltpu.sync_copy(zeros, acc.at[pl.ds(t * ZR, ZR)])
    plsc.subcore_barrier()

    base0 = t * ECH

    def do_group(goff, sizes):
        cps = []
        for k, sz in enumerate(sizes):
            off = goff + k * GK
            cps.append(pltpu.async_copy(he.at[pl.ds(off, sz)],
                                        rbuf.at[k, pl.ds(0, sz)], gsem))
            cps.append(pltpu.async_copy(dst.at[pl.ds(off, sz)],
                                        dbuf.at[k, pl.ds(0, sz)], isem))
        for cp in cps:
            cp.wait()
        for k, sz in enumerate(sizes):
            for j in range(sz // 16):
                v = dbuf[k, pl.ds(j * 16, 16)]
                inb = (v >= lo) & (v < lo + HALF)
                lbufs[k][pl.ds(j * 16, 16)] = jnp.where(inb, v - lo, DUMP)
        cps = []
        for k, sz in enumerate(sizes):
            cps.append(pltpu.async_copy(rbuf.at[k, pl.ds(0, sz)],
                                        acc.at[lbufs[k].at[pl.ds(0, sz)]],
                                        ssem, add=True))
        for cp in cps:
            cp.wait()

    def grp(g, _):
        do_group(base0 + g * (SKG * GK), [GK] * SKG)
        return _

    lax.fori_loop(0, SNGRP, grp, 0)
    do_group(base0 + SNGRP * SKG * GK, [STAIL])

    plsc.subcore_barrier()
    pltpu.sync_copy(acc.at[pl.ds(t * WOUT, WOUT)],
                    h_out.at[pl.ds(lo + t * WOUT, WOUT)])

    @pl.when(t == 0)
    def _():
        pltpu.sync_copy(acc.at[pl.ds(NS * WOUT, HALF - NS * WOUT)],
                        h_out.at[pl.ds(lo + NS * WOUT, HALF - NS * WOUT)])


def _sc_scatter(he, dst, zeros):
    f = pl.kernel(
        _scatter_body,
        out_type=jax.ShapeDtypeStruct((N, DIM), jnp.float32),
        mesh=_mesh(),
        scratch_types=[
            pltpu.VMEM((SKG, GK), jnp.int32),
            pltpu.VMEM((GK,), jnp.int32),
            pltpu.VMEM((GK,), jnp.int32),
            pltpu.VMEM((GK,), jnp.int32),
            pltpu.VMEM((SKG, GK, DIM), jnp.float32),
            pltpu.VMEM_SHARED((ACC_R, DIM), jnp.float32),
            pltpu.SemaphoreType.DMA,
            pltpu.SemaphoreType.DMA,
            pltpu.SemaphoreType.DMA,
        ],
        compiler_params=pltpu.CompilerParams(use_tc_tiling_on_sc=False),
    )
    return f(he, dst, zeros)


BN = 2048
BE = 2048
NTP = 128


def _embed_body(at_ref, emb_ref, out_ref):
    at = at_ref[...]
    tid = lax.broadcasted_iota(jnp.int32, (BN, NTP), 1)
    onehot = jnp.where(at == tid, 1.0, 0.0).astype(jnp.float32)
    out_ref[...] = jnp.dot(onehot, emb_ref[...],
                           preferred_element_type=jnp.float32)


def _tc_embed(atom_types, emb_pad):
    return pl.pallas_call(
        _embed_body,
        grid=(pl.cdiv(N, BN),),
        in_specs=[
            pl.BlockSpec((BN, 1), lambda i: (i, 0)),
            pl.BlockSpec((NTP, DIM), lambda i: (0, 0)),
        ],
        out_specs=pl.BlockSpec((BN, DIM), lambda i: (i, 0)),
        out_shape=jax.ShapeDtypeStruct((N, DIM), jnp.float32),
    )(atom_types.reshape(N, 1), emb_pad)


def _proj_body(nf_ref, wa_ref, wb_ref, q1_ref, q2_ref):
    nf = nf_ref[...]
    q1_ref[...] = jnp.dot(nf, wa_ref[...], preferred_element_type=jnp.float32)
    q2_ref[...] = jnp.dot(nf, wb_ref[...], preferred_element_type=jnp.float32)


def _tc_proj(nf, wa, wb):
    wspec = lambda r, c: pl.BlockSpec((r, c), lambda i: (0, 0))
    return pl.pallas_call(
        _proj_body,
        grid=(pl.cdiv(N, BN),),
        in_specs=[
            pl.BlockSpec((BN, DIM), lambda i: (i, 0)),
            wspec(DIM, HID), wspec(DIM, HID),
        ],
        out_specs=[pl.BlockSpec((BN, HID), lambda i: (i, 0)),
                   pl.BlockSpec((BN, HID), lambda i: (i, 0))],
        out_shape=[jax.ShapeDtypeStruct((N, HID), jnp.float32),
                   jax.ShapeDtypeStruct((N, HID), jnp.float32)],
    )(nf, wa, wb)


def _edge_body(first, last, qs_ref, qd_ref, ef_ref, w1c_ref, b1_ref, w2_ref,
               b2_ref, pw1_ref, pb1_ref, pw2_ref, pb2_ref, *out_refs):
    if first:
        d = ef_ref[...]
        step = CUTOFF / (RBF - 1)
        offs = lax.broadcasted_iota(jnp.int32, (1, RBF), 1).astype(jnp.float32) * step
        coeff = -0.5 / (step * step)
        ef = jnp.exp(coeff * (d - offs) ** 2)
    else:
        ef = ef_ref[...]
    t = qs_ref[...] + qd_ref[...]
    t += jnp.dot(ef, w1c_ref[...], preferred_element_type=jnp.float32)
    t = _ssp(t + b1_ref[...])
    ef_new = jnp.dot(t, w2_ref[...], preferred_element_type=jnp.float32) + b2_ref[...]
    u = _ssp(jnp.dot(ef_new, pw1_ref[...], preferred_element_type=jnp.float32)
             + pb1_ref[...])
    he = jnp.dot(u, pw2_ref[...], preferred_element_type=jnp.float32) + pb2_ref[...]
    if last:
        out_refs[0][...] = he
    else:
        out_refs[0][...] = ef_new
        out_refs[1][...] = he


def _tc_edge(qs, qd, ef, w1c, b1, w2, b2, pw1, pb1, pw2, pb2, first, last):
    efw = 1 if first else RBF
    wspec = lambda r, c: pl.BlockSpec((r, c), lambda i: (0, 0))
    outs = [jax.ShapeDtypeStruct((E, RBF), jnp.float32)]
    out_specs = [pl.BlockSpec((BE, RBF), lambda i: (i, 0))]
    if not last:
        outs.append(jax.ShapeDtypeStruct((E, RBF), jnp.float32))
        out_specs.append(pl.BlockSpec((BE, RBF), lambda i: (i, 0)))
    res = pl.pallas_call(
        functools.partial(_edge_body, first, last),
        grid=(pl.cdiv(E, BE),),
        in_specs=[
            pl.BlockSpec((BE, HID), lambda i: (i, 0)),
            pl.BlockSpec((BE, HID), lambda i: (i, 0)),
            pl.BlockSpec((BE, efw), lambda i: (i, 0)),
            wspec(RBF, 2 * DIM),
            wspec(1, 2 * DIM),
            wspec(2 * DIM, RBF),
            wspec(1, RBF),
            wspec(RBF, DIM),
            wspec(1, DIM),
            wspec(DIM, DIM),
            wspec(1, DIM),
        ],
        out_specs=out_specs,
        out_shape=outs,
    )(qs, qd, ef, w1c, b1, w2, b2, pw1, pb1, pw2, pb2)
    if last:
        return None, res[0]
    return res[0], res[1]


def _node_body(h_ref, nf_ref, w1_ref, b1_ref, w2_ref, b2_ref, out_ref):
    t = _ssp(jnp.dot(h_ref[...], w1_ref[...], preferred_element_type=jnp.float32)
             + b1_ref[...])
    out_ref[...] = (nf_ref[...] + b2_ref[...]
                    + jnp.dot(t, w2_ref[...], preferred_element_type=jnp.float32))


def _tc_node(h, nf, w1, b1, w2, b2):
    wspec = lambda r, c: pl.BlockSpec((r, c), lambda i: (0, 0))
    return pl.pallas_call(
        _node_body,
        grid=(pl.cdiv(N, BN),),
        in_specs=[
            pl.BlockSpec((BN, DIM), lambda i: (i, 0)),
            pl.BlockSpec((BN, DIM), lambda i: (i, 0)),
            wspec(DIM, DIM), wspec(1, DIM), wspec(DIM, DIM), wspec(1, DIM),
        ],
        out_specs=pl.BlockSpec((BN, DIM), lambda i: (i, 0)),
        out_shape=jax.ShapeDtypeStruct((N, DIM), jnp.float32),
    )(h, nf, w1, b1, w2, b2)


def _node_last_body(ngrid, h_ref, nf_ref, w1_ref, b1_ref, w2_ref, b2_ref,
                    d1w_ref, d1b_ref, d2w_ref, d2b_ref, out_ref):
    i = pl.program_id(0)
    t = _ssp(jnp.dot(h_ref[...], w1_ref[...], preferred_element_type=jnp.float32)
             + b1_ref[...])
    nf = (nf_ref[...] + b2_ref[...]
          + jnp.dot(t, w2_ref[...], preferred_element_type=jnp.float32))
    atom = _ssp(jnp.dot(nf, d1w_ref[...], preferred_element_type=jnp.float32)
                + d1b_ref[...])
    row = i * BN + lax.broadcasted_iota(jnp.int32, (BN, 1), 0)
    atom = jnp.where(row < N, atom, 0.0)
    partial = jnp.sum(atom * d2w_ref[...])

    @pl.when(i == 0)
    def _():
        out_ref[...] = jnp.zeros_like(out_ref)

    out_ref[...] += partial.reshape(1, 1)

    @pl.when(i == ngrid - 1)
    def _():
        out_ref[...] = out_ref[...] * (1.0 / N) + d2b_ref[...]


def _tc_node_last(h, nf, w1, b1, w2, b2, d1w, d1b, d2w, d2b):
    ngrid = pl.cdiv(N, BN)
    wspec = lambda r, c: pl.BlockSpec((r, c), lambda i: (0, 0))
    return pl.pallas_call(
        functools.partial(_node_last_body, ngrid),
        grid=(ngrid,),
        in_specs=[
            pl.BlockSpec((BN, DIM), lambda i: (i, 0)),
            pl.BlockSpec((BN, DIM), lambda i: (i, 0)),
            wspec(DIM, DIM), wspec(1, DIM), wspec(DIM, DIM), wspec(1, DIM),
            wspec(DIM, DIM // 2), wspec(1, DIM // 2),
            wspec(1, DIM // 2), wspec(1, 1),
        ],
        out_specs=pl.BlockSpec((1, 1), lambda i: (0, 0)),
        out_shape=jax.ShapeDtypeStruct((1, 1), jnp.float32),
    )(h, nf, w1, b1, w2, b2, d1w, d1b, d2w, d2b)


def kernel(atom_types, edge_index, distance, emb_table, eu_W1, eu_b1, eu_W2,
           eu_b2, pn1_W, pn1_b, pe_W1, pe_b1, pe_W2, pe_b2, pn2_W1, pn2_b1,
           pn2_W2, pn2_b2, d1_W, d1_b, d2_W, d2_b):
    src = edge_index[0]
    dst = edge_index[1]
    dist = distance.reshape(E, 1)
    emb_pad = jnp.pad(emb_table, ((0, NTP - NTYPES), (0, 0)))
    zeros = jnp.zeros((ZR, DIM), jnp.float32)

    nf = _tc_embed(atom_types, emb_pad)
    ef = dist
    for i in range(NCONV):
        first = i == 0
        last = i == NCONV - 1
        q1, q2 = _tc_proj(nf, eu_W1[i][:DIM], eu_W1[i][DIM:2 * DIM])
        qs, qd = _sc_gather(q1, q2, src, dst)
        ef, he = _tc_edge(qs, qd, ef, eu_W1[i][2 * DIM:],
                          eu_b1[i].reshape(1, -1),
                          eu_W2[i], eu_b2[i].reshape(1, -1),
                          pe_W1[i], pe_b1[i].reshape(1, -1),
                          pe_W2[i], pe_b2[i].reshape(1, -1), first, last)
        h = _sc_scatter(he, dst, zeros)
        if not last:
            nf = _tc_node(h, nf, pn2_W1[i], pn2_b1[i].reshape(1, -1),
                          pn2_W2[i], pn2_b2[i].reshape(1, -1))
        else:
            out = _tc_node_last(h, nf, pn2_W1[i], pn2_b1[i].reshape(1, -1),
                                pn2_W2[i], pn2_b2[i].reshape(1, -1),
                                d1_W, d1_b.reshape(1, -1),
                                d2_W.reshape(1, -1), d2_b.reshape(1, 1))
    return out


# speedup vs baseline: 2.2479x; 1.0467x over previous
import functools

import jax
import jax.numpy as jnp
from jax import lax
from jax.experimental import pallas as pl
from jax.experimental.pallas import tpu as pltpu
from jax.experimental.pallas import tpu_sc as plsc

N = 50000
E = 800000
DIM = 64
RBF = 64
NCONV = 3
CUTOFF = 5.0
NTYPES = 100

NC = 2
NS = 16
NW = NC * NS

_LOG2 = 0.6931471805599453


_LOG2E = 1.4426950408889634


def _sp(x):
    e = jnp.exp(-jnp.abs(x))
    return jnp.maximum(x, 0.0) + jnp.log(1.0 + e)


def _mesh():
    return plsc.VectorSubcoreMesh(
        core_axis_name="c", subcore_axis_name="s", num_cores=NC, num_subcores=NS
    )


EW = E // NW
GK = 128
QGK = 64
QKG = 3
QFULL = EW // QGK
QTAIL = EW - QFULL * QGK
QNGRP = QFULL // QKG
HID = 2 * DIM


def _gather_body(q1, q2, src, dst, qs, qd, ibs, ibd, rbs, rbd,
                 isem, gsem, wsem):
    wid = lax.axis_index("s") * NC + lax.axis_index("c")
    base0 = wid * EW

    def do_group(goff, sizes):
        cps = []
        for k, sz in enumerate(sizes):
            off = goff + k * QGK
            cps.append(pltpu.async_copy(src.at[pl.ds(off, sz)],
                                        ibs.at[k, pl.ds(0, sz)], isem))
            cps.append(pltpu.async_copy(dst.at[pl.ds(off, sz)],
                                        ibd.at[k, pl.ds(0, sz)], isem))
        for cp in cps:
            cp.wait()
        cps = []
        for k, sz in enumerate(sizes):
            cps.append(pltpu.async_copy(q1.at[ibs.at[k, pl.ds(0, sz)]],
                                        rbs.at[k, pl.ds(0, sz)], gsem))
            cps.append(pltpu.async_copy(q2.at[ibd.at[k, pl.ds(0, sz)]],
                                        rbd.at[k, pl.ds(0, sz)], gsem))
        for cp in cps:
            cp.wait()
        cps = []
        for k, sz in enumerate(sizes):
            off = goff + k * QGK
            cps.append(pltpu.async_copy(rbs.at[k, pl.ds(0, sz)],
                                        qs.at[pl.ds(off, sz)], wsem))
            cps.append(pltpu.async_copy(rbd.at[k, pl.ds(0, sz)],
                                        qd.at[pl.ds(off, sz)], wsem))
        for cp in cps:
            cp.wait()

    def grp(g, _):
        do_group(base0 + g * (QKG * QGK), [QGK] * QKG)
        return _

    lax.fori_loop(0, QNGRP, grp, 0)
    do_group(base0 + QNGRP * QKG * QGK, [QTAIL])


def _sc_gather(q1, q2, src, dst):
    f = pl.kernel(
        _gather_body,
        out_type=[
            jax.ShapeDtypeStruct((E, HID), jnp.float32),
            jax.ShapeDtypeStruct((E, HID), jnp.float32),
        ],
        mesh=_mesh(),
        scratch_types=[
            pltpu.VMEM((QKG, QGK), jnp.int32),
            pltpu.VMEM((QKG, QGK), jnp.int32),
            pltpu.VMEM((QKG, QGK, HID), jnp.float32),
            pltpu.VMEM((QKG, QGK, HID), jnp.float32),
            pltpu.SemaphoreType.DMA,
            pltpu.SemaphoreType.DMA,
            pltpu.SemaphoreType.DMA,
        ],
    )
    return f(q1, q2, src, dst)


HALF = N // NC
ECH = E // NS
SGK = 128
SKG = 3
SNGRP = ECH // (SKG * SGK)
STAIL = ECH - SNGRP * SKG * SGK
ZR = 1568
DUMP = NS * ZR
ACC_R = DUMP + 8
WOUT = 1560


def _scatter_body(he, dst, zeros, h_out, dbuf, lb0, lb1, lb2,
                  rbuf, acc, isem, gsem, ssem):
    c = lax.axis_index("c")
    t = lax.axis_index("s")
    lo = c * HALF
    lbufs = [lb0, lb1, lb2]

    pltpu.sync_copy(zeros, acc.at[pl.ds(t * ZR, ZR)])
    plsc.subcore_barrier()

    base0 = t * ECH

    def do_group(goff, sizes):
        cps = []
        for k, sz in enumerate(sizes):
            off = goff + k * SGK
            cps.append(pltpu.async_copy(he.at[pl.ds(off, sz)],
                                        rbuf.at[k, pl.ds(0, sz)], gsem))
            cps.append(pltpu.async_copy(dst.at[pl.ds(off, sz)],
                                        dbuf.at[k, pl.ds(0, sz)], isem))
        for cp in cps:
            cp.wait()
        for k, sz in enumerate(sizes):
            for j in range(sz // 16):
                v = dbuf[k, pl.ds(j * 16, 16)]
                inb = (v >= lo) & (v < lo + HALF)
                lbufs[k][pl.ds(j * 16, 16)] = jnp.where(inb, v - lo, DUMP)
        cps = []
        for k, sz in enumerate(sizes):
            cps.append(pltpu.async_copy(rbuf.at[k, pl.ds(0, sz)],
                                        acc.at[lbufs[k].at[pl.ds(0, sz)]],
                                        ssem, add=True))
        for cp in cps:
            cp.wait()

    def grp(g, _):
        do_group(base0 + g * (SKG * SGK), [SGK] * SKG)
        return _

    lax.fori_loop(0, SNGRP, grp, 0)
    do_group(base0 + SNGRP * SKG * SGK, [STAIL])

    plsc.subcore_barrier()
    pltpu.sync_copy(acc.at[pl.ds(t * WOUT, WOUT)],
                    h_out.at[pl.ds(lo + t * WOUT, WOUT)])

    @pl.when(t == 0)
    def _():
        pltpu.sync_copy(acc.at[pl.ds(NS * WOUT, HALF - NS * WOUT)],
                        h_out.at[pl.ds(lo + NS * WOUT, HALF - NS * WOUT)])


def _sc_scatter(he, dst, zeros):
    f = pl.kernel(
        _scatter_body,
        out_type=jax.ShapeDtypeStruct((N, DIM), jnp.float32),
        mesh=_mesh(),
        scratch_types=[
            pltpu.VMEM((SKG, SGK), jnp.int32),
            pltpu.VMEM((SGK,), jnp.int32),
            pltpu.VMEM((SGK,), jnp.int32),
            pltpu.VMEM((SGK,), jnp.int32),
            pltpu.VMEM((SKG, SGK, DIM), jnp.float32),
            pltpu.VMEM_SHARED((ACC_R, DIM), jnp.float32),
            pltpu.SemaphoreType.DMA,
            pltpu.SemaphoreType.DMA,
            pltpu.SemaphoreType.DMA,
        ],
        compiler_params=pltpu.CompilerParams(use_tc_tiling_on_sc=False),
    )
    return f(he, dst, zeros)


BN = 2048
BE = 2048
NTP = 128


def _embed_body(at_ref, emb_ref, out_ref):
    at = at_ref[...]
    tid = lax.broadcasted_iota(jnp.int32, (BN, NTP), 1)
    onehot = jnp.where(at == tid, 1.0, 0.0).astype(jnp.float32)
    out_ref[...] = jnp.dot(onehot, emb_ref[...],
                           preferred_element_type=jnp.float32)


def _tc_embed(atom_types, emb_pad):
    return pl.pallas_call(
        _embed_body,
        grid=(pl.cdiv(N, BN),),
        in_specs=[
            pl.BlockSpec((BN, 1), lambda i: (i, 0)),
            pl.BlockSpec((NTP, DIM), lambda i: (0, 0)),
        ],
        out_specs=pl.BlockSpec((BN, DIM), lambda i: (i, 0)),
        out_shape=jax.ShapeDtypeStruct((N, DIM), jnp.float32),
    )(atom_types.reshape(N, 1), emb_pad)


def _proj_body(nf_ref, wa_ref, wb_ref, q1_ref, q2_ref):
    nf = nf_ref[...]
    q1_ref[...] = jnp.dot(nf, wa_ref[...], preferred_element_type=jnp.float32)
    q2_ref[...] = jnp.dot(nf, wb_ref[...], preferred_element_type=jnp.float32)


def _tc_proj(nf, wa, wb):
    wspec = lambda r, c: pl.BlockSpec((r, c), lambda i: (0, 0))
    return pl.pallas_call(
        _proj_body,
        grid=(pl.cdiv(N, BN),),
        in_specs=[
            pl.BlockSpec((BN, DIM), lambda i: (i, 0)),
            wspec(DIM, HID), wspec(DIM, HID),
        ],
        out_specs=[pl.BlockSpec((BN, HID), lambda i: (i, 0)),
                   pl.BlockSpec((BN, HID), lambda i: (i, 0))],
        out_shape=[jax.ShapeDtypeStruct((N, HID), jnp.float32),
                   jax.ShapeDtypeStruct((N, HID), jnp.float32)],
    )(nf, wa, wb)


def _edge_body(first, last, qs_ref, qd_ref, ef_ref, w1c_ref, b1_ref, w2_ref,
               b2_ref, pw1_ref, pb1_ref, pw2_ref, pb2_ref, *out_refs):
    if first:
        d = ef_ref[...]
        step = CUTOFF / (RBF - 1)
        offs = lax.broadcasted_iota(jnp.int32, (1, RBF), 1).astype(jnp.float32) * step
        coeff = -0.5 / (step * step)
        ef = jnp.exp(coeff * (d - offs) ** 2)
    else:
        ef = ef_ref[...]
    t = qs_ref[...] + qd_ref[...]
    t += jnp.dot(ef, w1c_ref[...], preferred_element_type=jnp.float32)
    t = _sp(t + b1_ref[...])
    ef_new = jnp.dot(t, w2_ref[...], preferred_element_type=jnp.float32) + b2_ref[...]
    u = _sp(jnp.dot(ef_new, pw1_ref[...], preferred_element_type=jnp.float32)
             + pb1_ref[...])
    he = jnp.dot(u, pw2_ref[...], preferred_element_type=jnp.float32) + pb2_ref[...]
    if last:
        out_refs[0][...] = he
    else:
        out_refs[0][...] = ef_new
        out_refs[1][...] = he


def _tc_edge(qs, qd, ef, w1c, b1, w2, b2, pw1, pb1, pw2, pb2, first, last):
    efw = 1 if first else RBF
    wspec = lambda r, c: pl.BlockSpec((r, c), lambda i: (0, 0))
    outs = [jax.ShapeDtypeStruct((E, RBF), jnp.float32)]
    out_specs = [pl.BlockSpec((BE, RBF), lambda i: (i, 0))]
    if not last:
        outs.append(jax.ShapeDtypeStruct((E, RBF), jnp.float32))
        out_specs.append(pl.BlockSpec((BE, RBF), lambda i: (i, 0)))
    res = pl.pallas_call(
        functools.partial(_edge_body, first, last),
        grid=(pl.cdiv(E, BE),),
        in_specs=[
            pl.BlockSpec((BE, HID), lambda i: (i, 0)),
            pl.BlockSpec((BE, HID), lambda i: (i, 0)),
            pl.BlockSpec((BE, efw), lambda i: (i, 0)),
            wspec(RBF, 2 * DIM),
            wspec(1, 2 * DIM),
            wspec(2 * DIM, RBF),
            wspec(1, RBF),
            wspec(RBF, DIM),
            wspec(1, DIM),
            wspec(DIM, DIM),
            wspec(1, DIM),
        ],
        out_specs=out_specs,
        out_shape=outs,
    )(qs, qd, ef, w1c, b1, w2, b2, pw1, pb1, pw2, pb2)
    if last:
        return None, res[0]
    return res[0], res[1]


def _node_body(h_ref, nf_ref, w1_ref, b1_ref, w2_ref, b2_ref, out_ref):
    t = _sp(jnp.dot(h_ref[...], w1_ref[...], preferred_element_type=jnp.float32)
             + b1_ref[...])
    out_ref[...] = (nf_ref[...] + b2_ref[...]
                    + jnp.dot(t, w2_ref[...], preferred_element_type=jnp.float32))


def _tc_node(h, nf, w1, b1, w2, b2):
    wspec = lambda r, c: pl.BlockSpec((r, c), lambda i: (0, 0))
    return pl.pallas_call(
        _node_body,
        grid=(pl.cdiv(N, BN),),
        in_specs=[
            pl.BlockSpec((BN, DIM), lambda i: (i, 0)),
            pl.BlockSpec((BN, DIM), lambda i: (i, 0)),
            wspec(DIM, DIM), wspec(1, DIM), wspec(DIM, DIM), wspec(1, DIM),
        ],
        out_specs=pl.BlockSpec((BN, DIM), lambda i: (i, 0)),
        out_shape=jax.ShapeDtypeStruct((N, DIM), jnp.float32),
    )(h, nf, w1, b1, w2, b2)


def _node_last_body(ngrid, h_ref, nf_ref, w1_ref, b1_ref, w2_ref, b2_ref,
                    d1w_ref, d1b_ref, d2w_ref, d2b_ref, out_ref):
    i = pl.program_id(0)
    t = _sp(jnp.dot(h_ref[...], w1_ref[...], preferred_element_type=jnp.float32)
             + b1_ref[...])
    nf = (nf_ref[...] + b2_ref[...]
          + jnp.dot(t, w2_ref[...], preferred_element_type=jnp.float32))
    atom = _sp(jnp.dot(nf, d1w_ref[...], preferred_element_type=jnp.float32)
                + d1b_ref[...])
    row = i * BN + lax.broadcasted_iota(jnp.int32, (BN, 1), 0)
    atom = jnp.where(row < N, atom, 0.0)
    partial = jnp.sum(atom * d2w_ref[...])

    @pl.when(i == 0)
    def _():
        out_ref[...] = jnp.zeros_like(out_ref)

    out_ref[...] += partial.reshape(1, 1)

    @pl.when(i == ngrid - 1)
    def _():
        out_ref[...] = out_ref[...] * (1.0 / N) + d2b_ref[...]


def _tc_node_last(h, nf, w1, b1, w2, b2, d1w, d1b, d2w, d2b):
    ngrid = pl.cdiv(N, BN)
    wspec = lambda r, c: pl.BlockSpec((r, c), lambda i: (0, 0))
    return pl.pallas_call(
        functools.partial(_node_last_body, ngrid),
        grid=(ngrid,),
        in_specs=[
            pl.BlockSpec((BN, DIM), lambda i: (i, 0)),
            pl.BlockSpec((BN, DIM), lambda i: (i, 0)),
            wspec(DIM, DIM), wspec(1, DIM), wspec(DIM, DIM), wspec(1, DIM),
            wspec(DIM, DIM // 2), wspec(1, DIM // 2),
            wspec(1, DIM // 2), wspec(1, 1),
        ],
        out_specs=pl.BlockSpec((1, 1), lambda i: (0, 0)),
        out_shape=jax.ShapeDtypeStruct((1, 1), jnp.float32),
    )(h, nf, w1, b1, w2, b2, d1w, d1b, d2w, d2b)


def kernel(atom_types, edge_index, distance, emb_table, eu_W1, eu_b1, eu_W2,
           eu_b2, pn1_W, pn1_b, pe_W1, pe_b1, pe_W2, pe_b2, pn2_W1, pn2_b1,
           pn2_W2, pn2_b2, d1_W, d1_b, d2_W, d2_b):
    src = edge_index[0]
    dst = edge_index[1]
    dist = distance.reshape(E, 1)
    emb_pad = jnp.pad(emb_table, ((0, NTP - NTYPES), (0, 0)))
    zeros = jnp.zeros((ZR, DIM), jnp.float32)
    eu_b2e = eu_b2 - _LOG2 * eu_W2.sum(axis=1)
    pe_b2e = pe_b2 - _LOG2 * pe_W2.sum(axis=1)
    pn2_b2e = pn2_b2 - _LOG2 * pn2_W2.sum(axis=1)
    d2_be = d2_b - _LOG2 * d2_W.sum(axis=0)

    nf = _tc_embed(atom_types, emb_pad)
    ef = dist
    for i in range(NCONV):
        first = i == 0
        last = i == NCONV - 1
        q1, q2 = _tc_proj(nf, eu_W1[i][:DIM], eu_W1[i][DIM:2 * DIM])
        qs, qd = _sc_gather(q1, q2, src, dst)
        ef, he = _tc_edge(qs, qd, ef, eu_W1[i][2 * DIM:],
                          eu_b1[i].reshape(1, -1),
                          eu_W2[i], eu_b2e[i].reshape(1, -1),
                          pe_W1[i], pe_b1[i].reshape(1, -1),
                          pe_W2[i], pe_b2e[i].reshape(1, -1), first, last)
        h = _sc_scatter(he, dst, zeros)
        if not last:
            nf = _tc_node(h, nf, pn2_W1[i], pn2_b1[i].reshape(1, -1),
                          pn2_W2[i], pn2_b2e[i].reshape(1, -1))
        else:
            out = _tc_node_last(h, nf, pn2_W1[i], pn2_b1[i].reshape(1, -1),
                                pn2_W2[i], pn2_b2e[i].reshape(1, -1),
                                d1_W, d1_b.reshape(1, -1),
                                d2_W.reshape(1, -1), d2_be.reshape(1, 1))
    return out
